# baseline (speedup 1.0000x reference)
import numpy as np
import jax
import jax.numpy as jnp
from jax import lax
from jax.experimental import pallas as pl
from jax.experimental.pallas import tpu as pltpu

N = 4096
D = 256
H = 128
EMB = 64
C = 40
EPS = 1e-06
BM = 256
NB = N // BM


def _threefry_bits(lin_u32):
    k0 = np.uint32(0)
    k1 = np.uint32(1234)
    k2 = np.uint32(np.uint32(k0 ^ k1) ^ np.uint32(0x1BD11BDA))
    ks = (k0, k1, k2)
    rot = ((13, 15, 26, 6), (17, 29, 16, 24))
    x0 = jnp.full_like(lin_u32, k0)
    x1 = lin_u32 + k1
    for d in range(5):
        for r in rot[d % 2]:
            x0 = x0 + x1
            x1 = (x1 << np.uint32(r)) | (x1 >> np.uint32(32 - r))
            x1 = x1 ^ x0
        x0 = x0 + ks[(d + 1) % 3]
        x1 = x1 + ks[(d + 2) % 3] + np.uint32(d + 1)
    return x0 ^ x1


def _m1_kernel(adj_ref, feat_ref, wb_ref, wm_ref, m1_ref, fx1_ref):
    j = pl.program_id(0)

    @pl.when(j == 0)
    def _():
        fx1_ref[...] = jnp.dot(feat_ref[...], wb_ref[...],
                               preferred_element_type=jnp.float32)

    hid = jnp.dot(adj_ref[...], fx1_ref[...], preferred_element_type=jnp.float32)
    m1_ref[...] = jnp.dot(hid, wm_ref[...], preferred_element_type=jnp.float32)


def _z_kernel(adj_ref, m1_ref, z_ref):
    z_ref[...] = jnp.maximum(
        jnp.dot(adj_ref[...], m1_ref[...], preferred_element_type=jnp.float32), 0.0)


def _logits_kernel(zb_ref, z_ref, l_ref, mx_ref):
    j = pl.program_id(0)
    lb = lax.dot_general(zb_ref[...], z_ref[...], (((1,), (1,)), ((), ())),
                         preferred_element_type=jnp.float32)
    l_ref[...] = lb
    m = jnp.max(lb)

    @pl.when(j == 0)
    def _():
        mx_ref[0, 0] = m

    @pl.when(j > 0)
    def _():
        mx_ref[0, 0] = jnp.maximum(mx_ref[0, 0], m)


BN = 512
NBC = N // BN


def _sample_kernel(zr_ref, zc_ref, mx_ref, v8_ref):
    rb = pl.program_id(0)
    cb = pl.program_id(1)
    below = (cb + 1) * BN <= rb * BM

    @pl.when(below)
    def _():
        v8_ref[...] = jnp.zeros((BM, BN), jnp.int8)

    @pl.when(jnp.logical_not(below))
    def _():
        mx = mx_ref[0, 0]
        lb = lax.dot_general(zr_ref[...], zc_ref[...], (((1,), (1,)), ((), ())),
                             preferred_element_type=jnp.float32)
        p = jnp.clip(lb / mx, EPS, 1.0 - EPS)
        logit = jnp.log(p) - jnp.log1p(-p)
        r = lax.broadcasted_iota(jnp.int32, (BM, BN), 0) + rb * BM
        c = lax.broadcasted_iota(jnp.int32, (BM, BN), 1) + cb * BN
        lin = (r * N + c).astype(jnp.uint32)
        bits = _threefry_bits(lin)
        fb = (bits >> np.uint32(9)) | np.uint32(0x3F800000)
        fl = lax.bitcast_convert_type(fb, jnp.float32) - 1.0
        span = np.float32(np.float32(1.0 - EPS) - np.float32(EPS))
        u = jnp.maximum(np.float32(EPS), fl * span + np.float32(EPS))
        lnoise = jnp.log(u) - jnp.log1p(-u)
        soft = jax.nn.sigmoid(logit + lnoise)
        hard = jnp.round(soft)
        v8_ref[...] = jnp.where(c > r, hard, 0.0).astype(jnp.int8)


def _symt_kernel(va_ref, vb_ref, s8_ref, deg_ref):
    bi = pl.program_id(0)
    bj = pl.program_id(1)
    a = va_ref[...].astype(jnp.float32)
    b = vb_ref[...].astype(jnp.float32)
    rl = lax.broadcasted_iota(jnp.int32, (BM, BM), 0)
    cl = lax.broadcasted_iota(jnp.int32, (BM, BM), 1)
    eye = jnp.where(jnp.logical_and(rl == cl, bi == bj), 1.0, 0.0)
    s = a + b.T + eye
    s8_ref[...] = s.astype(jnp.int8)
    rs = jnp.sum(s, axis=1, keepdims=True)

    @pl.when(bj == 0)
    def _():
        deg_ref[...] = rs

    @pl.when(bj > 0)
    def _():
        deg_ref[...] = deg_ref[...] + rs


def _gcn1_kernel(s8_ref, feat_ref, w1_ref, b1_ref, deg_ref, h_ref, xs_ref):
    i = pl.program_id(0)

    @pl.when(i == 0)
    def _():
        x1 = jnp.dot(feat_ref[...], w1_ref[...], preferred_element_type=jnp.float32)
        dinv = 1.0 / jnp.sqrt(deg_ref[...])
        xs_ref[...] = dinv * x1

    s = s8_ref[...].astype(jnp.float32)
    y = jnp.dot(s, xs_ref[...], preferred_element_type=jnp.float32)
    dinvb = 1.0 / jnp.sqrt(deg_ref[pl.ds(i * BM, BM), :])
    h_ref[...] = jnp.maximum(dinvb * y + b1_ref[...], 0.0)


def _gcn2_kernel(s8_ref, h_ref, w2_ref, b2_ref, deg_ref, nc_ref, xs2_ref):
    i = pl.program_id(0)

    @pl.when(i == 0)
    def _():
        x2 = jnp.dot(h_ref[...], w2_ref[...], preferred_element_type=jnp.float32)
        dinv = 1.0 / jnp.sqrt(deg_ref[...])
        xs2_ref[...] = dinv * x2

    s = s8_ref[...].astype(jnp.float32)
    y = jnp.dot(s, xs2_ref[...], preferred_element_type=jnp.float32)
    dinvb = 1.0 / jnp.sqrt(deg_ref[pl.ds(i * BM, BM), :])
    nc_ref[...] = dinvb * y + b2_ref[...]


def kernel(adj_norm, adj_orig, features, W_base, W_mean, W1, b1, W2, b2):
    f32 = jnp.float32
    m1 = pl.pallas_call(
        _m1_kernel,
        grid=(NB,),
        in_specs=[pl.BlockSpec((BM, N), lambda j: (j, 0)),
                  pl.BlockSpec((N, D), lambda j: (0, 0)),
                  pl.BlockSpec((D, H), lambda j: (0, 0)),
                  pl.BlockSpec((H, EMB), lambda j: (0, 0))],
        out_specs=pl.BlockSpec((BM, EMB), lambda j: (j, 0)),
        out_shape=jax.ShapeDtypeStruct((N, EMB), f32),
        scratch_shapes=[pltpu.VMEM((N, H), f32)],
    )(adj_norm, features, W_base, W_mean)

    z = pl.pallas_call(
        _z_kernel,
        grid=(NB,),
        in_specs=[pl.BlockSpec((BM, N), lambda j: (j, 0)),
                  pl.BlockSpec((N, EMB), lambda j: (0, 0))],
        out_specs=pl.BlockSpec((BM, EMB), lambda j: (j, 0)),
        out_shape=jax.ShapeDtypeStruct((N, EMB), f32),
    )(adj_norm, m1)

    adj_logits, mx = pl.pallas_call(
        _logits_kernel,
        grid=(NB,),
        in_specs=[pl.BlockSpec((BM, EMB), lambda j: (j, 0)),
                  pl.BlockSpec((N, EMB), lambda j: (0, 0))],
        out_specs=[pl.BlockSpec((BM, N), lambda j: (j, 0)),
                   pl.BlockSpec((1, 1), lambda j: (0, 0), memory_space=pltpu.SMEM)],
        out_shape=[jax.ShapeDtypeStruct((N, N), f32),
                   jax.ShapeDtypeStruct((1, 1), f32)],
    )(z, z)

    v8 = pl.pallas_call(
        _sample_kernel,
        grid=(NB, NBC),
        in_specs=[pl.BlockSpec((BM, EMB), lambda rb, cb: (rb, 0)),
                  pl.BlockSpec((BN, EMB), lambda rb, cb: (cb, 0)),
                  pl.BlockSpec((1, 1), lambda rb, cb: (0, 0),
                               memory_space=pltpu.SMEM)],
        out_specs=pl.BlockSpec((BM, BN), lambda rb, cb: (rb, cb)),
        out_shape=jax.ShapeDtypeStruct((N, N), jnp.int8),
    )(z, z, mx)

    s8, deg = pl.pallas_call(
        _symt_kernel,
        grid=(NB, NB),
        in_specs=[pl.BlockSpec((BM, BM), lambda bi, bj: (bi, bj)),
                  pl.BlockSpec((BM, BM), lambda bi, bj: (bj, bi))],
        out_specs=[pl.BlockSpec((BM, BM), lambda bi, bj: (bi, bj)),
                   pl.BlockSpec((BM, 1), lambda bi, bj: (bi, 0))],
        out_shape=[jax.ShapeDtypeStruct((N, N), jnp.int8),
                   jax.ShapeDtypeStruct((N, 1), f32)],
    )(v8, v8)

    h = pl.pallas_call(
        _gcn1_kernel,
        grid=(NB,),
        in_specs=[pl.BlockSpec((BM, N), lambda i: (i, 0)),
                  pl.BlockSpec((N, D), lambda i: (0, 0)),
                  pl.BlockSpec((D, H), lambda i: (0, 0)),
                  pl.BlockSpec((1, H), lambda i: (0, 0)),
                  pl.BlockSpec((N, 1), lambda i: (0, 0))],
        out_specs=pl.BlockSpec((BM, H), lambda i: (i, 0)),
        out_shape=jax.ShapeDtypeStruct((N, H), f32),
        scratch_shapes=[pltpu.VMEM((N, H), f32)],
    )(s8, features, W1, b1.reshape(1, H), deg)

    nc_logits = pl.pallas_call(
        _gcn2_kernel,
        grid=(NB,),
        in_specs=[pl.BlockSpec((BM, N), lambda i: (i, 0)),
                  pl.BlockSpec((N, H), lambda i: (0, 0)),
                  pl.BlockSpec((H, C), lambda i: (0, 0)),
                  pl.BlockSpec((1, C), lambda i: (0, 0)),
                  pl.BlockSpec((N, 1), lambda i: (0, 0))],
        out_specs=pl.BlockSpec((BM, C), lambda i: (i, 0)),
        out_shape=jax.ShapeDtypeStruct((N, C), f32),
        scratch_shapes=[pltpu.VMEM((N, C), f32)],
    )(s8, h, W2, b2.reshape(1, C), deg)

    return nc_logits, adj_logits


# speedup vs baseline: 1.4891x; 1.4891x over previous
import numpy as np
import jax
import jax.numpy as jnp
from jax import lax
from jax.experimental import pallas as pl
from jax.experimental.pallas import tpu as pltpu

N = 4096
D = 256
H = 128
EMB = 64
C = 40
EPS = 1e-06
BM = 256
NB = N // BM


def _threefry_bits(lin_u32):
    k0 = np.uint32(0)
    k1 = np.uint32(1234)
    k2 = np.uint32(np.uint32(k0 ^ k1) ^ np.uint32(0x1BD11BDA))
    ks = (k0, k1, k2)
    rot = ((13, 15, 26, 6), (17, 29, 16, 24))
    x0 = jnp.full_like(lin_u32, k0)
    x1 = lin_u32 + k1
    for d in range(5):
        for r in rot[d % 2]:
            x0 = x0 + x1
            x1 = (x1 << np.uint32(r)) | (x1 >> np.uint32(32 - r))
            x1 = x1 ^ x0
        x0 = x0 + ks[(d + 1) % 3]
        x1 = x1 + ks[(d + 2) % 3] + np.uint32(d + 1)
    return x0 ^ x1


def _m1_kernel(adj_ref, feat_ref, wb_ref, wm_ref, m1_ref, fx1_ref):
    j = pl.program_id(0)

    @pl.when(j == 0)
    def _():
        fx1_ref[...] = jnp.dot(feat_ref[...], wb_ref[...],
                               preferred_element_type=jnp.float32)

    hid = jnp.dot(adj_ref[...], fx1_ref[...], preferred_element_type=jnp.float32)
    m1_ref[...] = jnp.dot(hid, wm_ref[...], preferred_element_type=jnp.float32)


def _z_kernel(adj_ref, m1_ref, z_ref):
    z_ref[...] = jnp.maximum(
        jnp.dot(adj_ref[...], m1_ref[...], preferred_element_type=jnp.float32), 0.0)


def _logits_kernel(zb_ref, z_ref, l_ref, mx_ref):
    j = pl.program_id(0)
    lb = lax.dot_general(zb_ref[...], z_ref[...], (((1,), (1,)), ((), ())),
                         preferred_element_type=jnp.float32)
    l_ref[...] = lb
    m = jnp.max(lb)

    @pl.when(j == 0)
    def _():
        mx_ref[0, 0] = m

    @pl.when(j > 0)
    def _():
        mx_ref[0, 0] = jnp.maximum(mx_ref[0, 0], m)


BK = 256
NBK = N // BK


def _sample_kernel(l_ref, mx_ref, v8_ref):
    rb = pl.program_id(0)
    rcp = 1.0 / mx_ref[0, 0]
    v8_ref[...] = jnp.zeros((BM, N), jnp.int8)
    i0 = lax.broadcasted_iota(jnp.int32, (BM, BK), 0)
    i1 = lax.broadcasted_iota(jnp.int32, (BM, BK), 1)
    gi = i0 * N + i1
    di = i1 - i0
    span = np.float32(np.float32(1.0 - EPS) - np.float32(EPS))

    def body(cb, carry):
        lb = l_ref[:, pl.ds(cb * BK, BK)]
        p = jnp.clip(lb * rcp, EPS, 1.0 - EPS)
        base = rb * (BM * N) + cb * BK
        bits = _threefry_bits((gi + base).astype(jnp.uint32))
        fb = (bits >> np.uint32(9)) | np.uint32(0x3F800000)
        fl = lax.bitcast_convert_type(fb, jnp.float32) - 1.0
        u = jnp.maximum(np.float32(EPS), fl * span + np.float32(EPS))
        upper = di > rb * BM - cb * BK
        keep = jnp.logical_and(p + u > 1.0, upper)
        v8_ref[:, pl.ds(cb * BK, BK)] = keep.astype(jnp.int8)
        return carry

    lax.fori_loop(rb, NBK, body, 0)


def _symt_kernel(va_ref, vb_ref, s8_ref, deg_ref):
    bi = pl.program_id(0)
    bj = pl.program_id(1)
    a = va_ref[...].astype(jnp.float32)
    b = vb_ref[...].astype(jnp.float32)
    rl = lax.broadcasted_iota(jnp.int32, (BM, BM), 0)
    cl = lax.broadcasted_iota(jnp.int32, (BM, BM), 1)
    eye = jnp.where(jnp.logical_and(rl == cl, bi == bj), 1.0, 0.0)
    s = a + b.T + eye
    s8_ref[...] = s.astype(jnp.int8)
    rs = jnp.sum(s, axis=1, keepdims=True)

    @pl.when(bj == 0)
    def _():
        deg_ref[...] = rs

    @pl.when(bj > 0)
    def _():
        deg_ref[...] = deg_ref[...] + rs


def _gcn1_kernel(s8_ref, feat_ref, w1_ref, b1_ref, deg_ref, h_ref, xs_ref):
    i = pl.program_id(0)

    @pl.when(i == 0)
    def _():
        x1 = jnp.dot(feat_ref[...], w1_ref[...], preferred_element_type=jnp.float32)
        dinv = 1.0 / jnp.sqrt(deg_ref[...])
        xs_ref[...] = dinv * x1

    s = s8_ref[...].astype(jnp.float32)
    y = jnp.dot(s, xs_ref[...], preferred_element_type=jnp.float32)
    dinvb = 1.0 / jnp.sqrt(deg_ref[pl.ds(i * BM, BM), :])
    h_ref[...] = jnp.maximum(dinvb * y + b1_ref[...], 0.0)


def _gcn2_kernel(s8_ref, h_ref, w2_ref, b2_ref, deg_ref, nc_ref, xs2_ref):
    i = pl.program_id(0)

    @pl.when(i == 0)
    def _():
        x2 = jnp.dot(h_ref[...], w2_ref[...], preferred_element_type=jnp.float32)
        dinv = 1.0 / jnp.sqrt(deg_ref[...])
        xs2_ref[...] = dinv * x2

    s = s8_ref[...].astype(jnp.float32)
    y = jnp.dot(s, xs2_ref[...], preferred_element_type=jnp.float32)
    dinvb = 1.0 / jnp.sqrt(deg_ref[pl.ds(i * BM, BM), :])
    nc_ref[...] = dinvb * y + b2_ref[...]


def kernel(adj_norm, adj_orig, features, W_base, W_mean, W1, b1, W2, b2):
    f32 = jnp.float32
    m1 = pl.pallas_call(
        _m1_kernel,
        grid=(NB,),
        in_specs=[pl.BlockSpec((BM, N), lambda j: (j, 0)),
                  pl.BlockSpec((N, D), lambda j: (0, 0)),
                  pl.BlockSpec((D, H), lambda j: (0, 0)),
                  pl.BlockSpec((H, EMB), lambda j: (0, 0))],
        out_specs=pl.BlockSpec((BM, EMB), lambda j: (j, 0)),
        out_shape=jax.ShapeDtypeStruct((N, EMB), f32),
        scratch_shapes=[pltpu.VMEM((N, H), f32)],
    )(adj_norm, features, W_base, W_mean)

    z = pl.pallas_call(
        _z_kernel,
        grid=(NB,),
        in_specs=[pl.BlockSpec((BM, N), lambda j: (j, 0)),
                  pl.BlockSpec((N, EMB), lambda j: (0, 0))],
        out_specs=pl.BlockSpec((BM, EMB), lambda j: (j, 0)),
        out_shape=jax.ShapeDtypeStruct((N, EMB), f32),
    )(adj_norm, m1)

    adj_logits, mx = pl.pallas_call(
        _logits_kernel,
        grid=(NB,),
        in_specs=[pl.BlockSpec((BM, EMB), lambda j: (j, 0)),
                  pl.BlockSpec((N, EMB), lambda j: (0, 0))],
        out_specs=[pl.BlockSpec((BM, N), lambda j: (j, 0)),
                   pl.BlockSpec((1, 1), lambda j: (0, 0), memory_space=pltpu.SMEM)],
        out_shape=[jax.ShapeDtypeStruct((N, N), f32),
                   jax.ShapeDtypeStruct((1, 1), f32)],
    )(z, z)

    v8 = pl.pallas_call(
        _sample_kernel,
        grid=(NB,),
        in_specs=[pl.BlockSpec((BM, N), lambda j: (j, 0)),
                  pl.BlockSpec((1, 1), lambda j: (0, 0), memory_space=pltpu.SMEM)],
        out_specs=pl.BlockSpec((BM, N), lambda j: (j, 0)),
        out_shape=jax.ShapeDtypeStruct((N, N), jnp.int8),
    )(adj_logits, mx)

    s8, deg = pl.pallas_call(
        _symt_kernel,
        grid=(NB, NB),
        in_specs=[pl.BlockSpec((BM, BM), lambda bi, bj: (bi, bj)),
                  pl.BlockSpec((BM, BM), lambda bi, bj: (bj, bi))],
        out_specs=[pl.BlockSpec((BM, BM), lambda bi, bj: (bi, bj)),
                   pl.BlockSpec((BM, 1), lambda bi, bj: (bi, 0))],
        out_shape=[jax.ShapeDtypeStruct((N, N), jnp.int8),
                   jax.ShapeDtypeStruct((N, 1), f32)],
    )(v8, v8)

    h = pl.pallas_call(
        _gcn1_kernel,
        grid=(NB,),
        in_specs=[pl.BlockSpec((BM, N), lambda i: (i, 0)),
                  pl.BlockSpec((N, D), lambda i: (0, 0)),
                  pl.BlockSpec((D, H), lambda i: (0, 0)),
                  pl.BlockSpec((1, H), lambda i: (0, 0)),
                  pl.BlockSpec((N, 1), lambda i: (0, 0))],
        out_specs=pl.BlockSpec((BM, H), lambda i: (i, 0)),
        out_shape=jax.ShapeDtypeStruct((N, H), f32),
        scratch_shapes=[pltpu.VMEM((N, H), f32)],
    )(s8, features, W1, b1.reshape(1, H), deg)

    nc_logits = pl.pallas_call(
        _gcn2_kernel,
        grid=(NB,),
        in_specs=[pl.BlockSpec((BM, N), lambda i: (i, 0)),
                  pl.BlockSpec((N, H), lambda i: (0, 0)),
                  pl.BlockSpec((H, C), lambda i: (0, 0)),
                  pl.BlockSpec((1, C), lambda i: (0, 0)),
                  pl.BlockSpec((N, 1), lambda i: (0, 0))],
        out_specs=pl.BlockSpec((BM, C), lambda i: (i, 0)),
        out_shape=jax.ShapeDtypeStruct((N, C), f32),
        scratch_shapes=[pltpu.VMEM((N, C), f32)],
    )(s8, h, W2, b2.reshape(1, C), deg)

    return nc_logits, adj_logits
